# Initial kernel scaffold; baseline (speedup 1.0000x reference)
import jax
import jax.numpy as jnp
from jax.experimental import pallas as pl


def kernel(embedding, adj, W_heads, a_heads, W_out, a_out):
    raise NotImplementedError("write your pallas kernel here")


# speedup vs baseline: 8.4236x; 8.4236x over previous
import functools

import jax
import jax.numpy as jnp
from jax import lax
from jax.experimental import pallas as pl
from jax.experimental.pallas import tpu as pltpu
from jax.experimental.pallas import tpu_sc as plsc

_NW = 32
_C = 4


def _matmul(x, w, bm):
    n, k = x.shape
    m = w.shape[1]

    def body(x_ref, w_ref, o_ref):
        o_ref[...] = jnp.dot(x_ref[...], w_ref[...],
                             preferred_element_type=jnp.float32)

    return pl.pallas_call(
        body,
        grid=(n // bm,),
        in_specs=[pl.BlockSpec((bm, k), lambda i: (i, 0)),
                  pl.BlockSpec((k, m), lambda i: (0, 0))],
        out_specs=pl.BlockSpec((bm, m), lambda i: (i, 0)),
        out_shape=jax.ShapeDtypeStruct((n, m), jnp.float32),
    )(x, w)


def _gat_sc_layer(NP, D, nheads, F, GW, apply_elu):
    npw = NP // _NW
    nchunks = npw // _C
    E = _C * D
    outw = nheads * F
    scol0 = nheads * F
    nacc = F // 16
    mesh = plsc.VectorSubcoreMesh(core_axis_name="c", subcore_axis_name="s")

    @functools.partial(
        pl.kernel,
        mesh=mesh,
        compiler_params=pltpu.CompilerParams(use_tc_tiling_on_sc=False,
                                             needs_layout_passes=False),
        out_type=jax.ShapeDtypeStruct((NP, outw), jnp.float32),
        scratch_types=[
            pltpu.VMEM((E,), jnp.int32),
            pltpu.VMEM((E, GW), jnp.float32),
            pltpu.VMEM((_C, outw), jnp.float32),
            pltpu.SemaphoreType.DMA,
        ],
    )
    def k(adjf, g, out, idx_v, rows_v, o_v, sem):
        wid = lax.axis_index("s") * 2 + lax.axis_index("c")
        base = wid * npw

        def chunk_body(c, carry):
            nb = base + c * _C
            pltpu.sync_copy(adjf.at[pl.ds(nb * D, E)], idx_v)
            pltpu.async_copy(g.at[idx_v], rows_v, sem).wait()

            def node_body(i, carry2):
                r0 = i * D
                stv = rows_v[r0, pl.ds(scol0, 16)]
                for h in range(nheads):
                    scol = scol0 + 2 * h
                    it = lax.iota(jnp.int32, 16)
                    cs = jnp.full((16,), scol, jnp.int32)
                    s0 = plsc.load_gather(rows_v, [r0 + it, cs])
                    s1 = plsc.load_gather(rows_v, [r0 + 16 + it, cs])
                    t = jnp.broadcast_to(stv[2 * h + 1], (16,))
                    e0 = s0 + t
                    e1 = s1 + t
                    e0 = jnp.maximum(e0, 0.2 * e0)
                    e1 = jnp.maximum(e1, 0.2 * e1)
                    m = jnp.maximum(jnp.max(e0), jnp.max(e1))
                    p0 = jnp.exp(e0 - m)
                    p1 = jnp.exp(e1 - m)
                    z = jnp.sum(p0) + jnp.sum(p1)
                    zrv = 1.0 / jnp.broadcast_to(z, (16,))
                    acc_a = [jnp.zeros((16,), jnp.float32) for _ in range(nacc)]
                    acc_b = [jnp.zeros((16,), jnp.float32) for _ in range(nacc)]
                    for d in range(0, D, 2):
                        pa = jnp.broadcast_to((p0 if d < 16 else p1)[d % 16], (16,))
                        pb = jnp.broadcast_to((p0 if d + 1 < 16 else p1)[(d + 1) % 16], (16,))
                        for fg in range(nacc):
                            col = h * F + fg * 16
                            acc_a[fg] = acc_a[fg] + pa * rows_v[r0 + d, pl.ds(col, 16)]
                            acc_b[fg] = acc_b[fg] + pb * rows_v[r0 + d + 1, pl.ds(col, 16)]
                    for fg in range(nacc):
                        acc = (acc_a[fg] + acc_b[fg]) * zrv
                        if apply_elu:
                            acc = jnp.where(acc > 0.0, acc, jnp.exp(acc) - 1.0)
                        o_v[i, pl.ds(h * F + fg * 16, 16)] = acc
                return carry2

            lax.fori_loop(0, _C, node_body, 0)
            pltpu.sync_copy(o_v, out.at[pl.ds(nb, _C)])
            return carry

        lax.fori_loop(0, nchunks, chunk_body, 0)

    return k


def _elu_logsoftmax(zin, n_valid):
    NPl, cls = zin.shape

    def body(z_ref, o_ref):
        zz = z_ref[...]
        x = jnp.where(zz > 0.0, zz, jnp.exp(zz) - 1.0)
        valid = lax.broadcasted_iota(jnp.int32, (NPl, cls), 0) < n_valid
        xm = jnp.where(valid, x, -jnp.inf)
        mx = jnp.max(xm, axis=0, keepdims=True)
        se = jnp.sum(jnp.exp(xm - mx), axis=0, keepdims=True)
        o_ref[...] = x - (mx + jnp.log(se))

    return pl.pallas_call(
        body,
        out_shape=jax.ShapeDtypeStruct((NPl, cls), jnp.float32),
    )(zin)


def kernel(embedding, adj, W_heads, a_heads, W_out, a_out):
    bs, N, nfeat = embedding.shape
    nheads, _, nhid = W_heads.shape
    D = adj.shape[2]
    nclass = W_out.shape[1]
    NP = -(-N // 1024) * 1024

    x = embedding.reshape(N, nfeat)
    xp = jnp.pad(x, ((0, NP - N), (0, 0)))
    adjf = jnp.pad(adj.reshape(N, D), ((0, NP - N), (0, 0))).reshape(NP * D)

    Wc = jnp.swapaxes(W_heads, 0, 1).reshape(nfeat, nheads * nhid)
    a1 = a_heads[:, :nhid, 0]
    a2 = a_heads[:, nhid:, 0]
    sW = jnp.einsum('hfk,hk->fh', W_heads, a1)
    tW = jnp.einsum('hfk,hk->fh', W_heads, a2)
    stW = jnp.stack([sW, tW], axis=2).reshape(nfeat, 2 * nheads)
    GW1 = 144
    M1 = jnp.concatenate(
        [Wc, stW,
         jnp.zeros((nfeat, GW1 - nheads * nhid - 2 * nheads), jnp.float32)],
        axis=1)
    G1 = _matmul(xp, M1, 512)

    l1 = _gat_sc_layer(NP, D, nheads, nhid, GW1, True)
    x1 = l1(adjf, G1)

    GW2 = 48
    M2 = jnp.concatenate(
        [W_out, W_out @ a_out[:nclass], W_out @ a_out[nclass:],
         jnp.zeros((nheads * nhid, GW2 - nclass - 2), jnp.float32)],
        axis=1)
    G2 = _matmul(x1, M2, 512)

    l2 = _gat_sc_layer(NP, D, 1, nclass, GW2, False)
    z = l2(adjf, G2)

    out = _elu_logsoftmax(z, N)
    return out[:N].reshape(bs, N, nclass)


# speedup vs baseline: 10.7476x; 1.2759x over previous
import functools

import jax
import jax.numpy as jnp
from jax import lax
from jax.experimental import pallas as pl
from jax.experimental.pallas import tpu as pltpu
from jax.experimental.pallas import tpu_sc as plsc

_NW = 32
_C = 4


def _matmul(x, w, bm):
    n, k = x.shape
    m = w.shape[1]

    def body(x_ref, w_ref, o_ref):
        o_ref[...] = jnp.dot(x_ref[...], w_ref[...],
                             preferred_element_type=jnp.float32)

    return pl.pallas_call(
        body,
        grid=(n // bm,),
        in_specs=[pl.BlockSpec((bm, k), lambda i: (i, 0)),
                  pl.BlockSpec((k, m), lambda i: (0, 0))],
        out_specs=pl.BlockSpec((bm, m), lambda i: (i, 0)),
        out_shape=jax.ShapeDtypeStruct((n, m), jnp.float32),
    )(x, w)


def _gat_sc_layer(NP, D, nheads, F, GW, apply_elu):
    npw = NP // _NW
    nchunks = npw // _C
    E = _C * D
    outw = nheads * F
    scol0 = nheads * F
    nacc = F // 16
    mesh = plsc.VectorSubcoreMesh(core_axis_name="c", subcore_axis_name="s")

    @functools.partial(
        pl.kernel,
        mesh=mesh,
        compiler_params=pltpu.CompilerParams(use_tc_tiling_on_sc=False,
                                             needs_layout_passes=False),
        out_type=jax.ShapeDtypeStruct((NP, outw), jnp.float32),
        scratch_types=[
            pltpu.VMEM((E,), jnp.int32),
            pltpu.VMEM((E,), jnp.int32),
            pltpu.VMEM((E, GW), jnp.float32),
            pltpu.VMEM((E, GW), jnp.float32),
            pltpu.VMEM((_C, outw), jnp.float32),
            pltpu.SemaphoreType.DMA,
            pltpu.SemaphoreType.DMA,
        ],
    )
    def k(adjf, g, out, idx0_v, idx1_v, rows0_v, rows1_v, o_v, sem0, sem1):
        wid = lax.axis_index("s") * 2 + lax.axis_index("c")
        base = wid * npw

        def start_gather(c, idx_v, rows_v, sem):
            nb = base + c * _C
            pltpu.sync_copy(adjf.at[pl.ds(nb * D, E)], idx_v)
            pltpu.make_async_copy(g.at[idx_v], rows_v, sem).start()

        def compute(c, rows_v):
            nb = base + c * _C

            def node_body(i, carry2):
                r0 = i * D
                stv = rows_v[r0, pl.ds(scol0, 16)]
                for h in range(nheads):
                    scol = scol0 + 2 * h
                    it = lax.iota(jnp.int32, 16)
                    cs = jnp.full((16,), scol, jnp.int32)
                    s0 = plsc.load_gather(rows_v, [r0 + it, cs])
                    s1 = plsc.load_gather(rows_v, [r0 + 16 + it, cs])
                    t = jnp.broadcast_to(stv[2 * h + 1], (16,))
                    e0 = s0 + t
                    e1 = s1 + t
                    e0 = jnp.maximum(e0, 0.2 * e0)
                    e1 = jnp.maximum(e1, 0.2 * e1)
                    m = jnp.maximum(jnp.max(e0), jnp.max(e1))
                    p0 = jnp.exp(e0 - m)
                    p1 = jnp.exp(e1 - m)
                    z = jnp.sum(p0) + jnp.sum(p1)
                    zrv = 1.0 / jnp.broadcast_to(z, (16,))
                    acc_a = [jnp.zeros((16,), jnp.float32) for _ in range(nacc)]
                    acc_b = [jnp.zeros((16,), jnp.float32) for _ in range(nacc)]
                    for d in range(0, D, 2):
                        pa = jnp.broadcast_to((p0 if d < 16 else p1)[d % 16], (16,))
                        pb = jnp.broadcast_to((p0 if d + 1 < 16 else p1)[(d + 1) % 16], (16,))
                        for fg in range(nacc):
                            col = h * F + fg * 16
                            acc_a[fg] = acc_a[fg] + pa * rows_v[r0 + d, pl.ds(col, 16)]
                            acc_b[fg] = acc_b[fg] + pb * rows_v[r0 + d + 1, pl.ds(col, 16)]
                    for fg in range(nacc):
                        acc = (acc_a[fg] + acc_b[fg]) * zrv
                        if apply_elu:
                            acc = jnp.where(acc > 0.0, acc, jnp.exp(acc) - 1.0)
                        o_v[i, pl.ds(h * F + fg * 16, 16)] = acc
                return carry2

            lax.fori_loop(0, _C, node_body, 0)
            pltpu.sync_copy(o_v, out.at[pl.ds(nb, _C)])

        npairs = nchunks // 2
        start_gather(0, idx0_v, rows0_v, sem0)

        def pair_body(gi, carry):
            a = 2 * gi
            start_gather(a + 1, idx1_v, rows1_v, sem1)
            pltpu.make_async_copy(g.at[idx0_v], rows0_v, sem0).wait()
            compute(a, rows0_v)

            @pl.when(gi < npairs - 1)
            def _():
                start_gather(a + 2, idx0_v, rows0_v, sem0)

            pltpu.make_async_copy(g.at[idx1_v], rows1_v, sem1).wait()
            compute(a + 1, rows1_v)
            return carry

        lax.fori_loop(0, npairs, pair_body, 0)

    return k


def _elu_logsoftmax(zin, n_valid):
    NPl, cls = zin.shape

    def body(z_ref, o_ref):
        zz = z_ref[...]
        x = jnp.where(zz > 0.0, zz, jnp.exp(zz) - 1.0)
        valid = lax.broadcasted_iota(jnp.int32, (NPl, cls), 0) < n_valid
        xm = jnp.where(valid, x, -jnp.inf)
        mx = jnp.max(xm, axis=0, keepdims=True)
        se = jnp.sum(jnp.exp(xm - mx), axis=0, keepdims=True)
        o_ref[...] = x - (mx + jnp.log(se))

    return pl.pallas_call(
        body,
        out_shape=jax.ShapeDtypeStruct((NPl, cls), jnp.float32),
    )(zin)


def kernel(embedding, adj, W_heads, a_heads, W_out, a_out):
    bs, N, nfeat = embedding.shape
    nheads, _, nhid = W_heads.shape
    D = adj.shape[2]
    nclass = W_out.shape[1]
    NP = -(-N // 1024) * 1024

    x = embedding.reshape(N, nfeat)
    xp = jnp.pad(x, ((0, NP - N), (0, 0)))
    adjf = jnp.pad(adj.reshape(N, D), ((0, NP - N), (0, 0))).reshape(NP * D)

    Wc = jnp.swapaxes(W_heads, 0, 1).reshape(nfeat, nheads * nhid)
    a1 = a_heads[:, :nhid, 0]
    a2 = a_heads[:, nhid:, 0]
    sW = jnp.einsum('hfk,hk->fh', W_heads, a1)
    tW = jnp.einsum('hfk,hk->fh', W_heads, a2)
    stW = jnp.stack([sW, tW], axis=2).reshape(nfeat, 2 * nheads)
    GW1 = 144
    M1 = jnp.concatenate(
        [Wc, stW,
         jnp.zeros((nfeat, GW1 - nheads * nhid - 2 * nheads), jnp.float32)],
        axis=1)
    G1 = _matmul(xp, M1, 512)

    l1 = _gat_sc_layer(NP, D, nheads, nhid, GW1, True)
    x1 = l1(adjf, G1)

    GW2 = 48
    M2 = jnp.concatenate(
        [W_out, W_out @ a_out[:nclass], W_out @ a_out[nclass:],
         jnp.zeros((nheads * nhid, GW2 - nclass - 2), jnp.float32)],
        axis=1)
    G2 = _matmul(x1, M2, 512)

    l2 = _gat_sc_layer(NP, D, 1, nclass, GW2, False)
    z = l2(adjf, G2)

    out = _elu_logsoftmax(z, N)
    return out[:N].reshape(bs, N, nclass)


# speedup vs baseline: 23.8163x; 2.2160x over previous
import functools

import jax
import jax.numpy as jnp
from jax import lax
from jax.experimental import pallas as pl
from jax.experimental.pallas import tpu as pltpu
from jax.experimental.pallas import tpu_sc as plsc

_NW = 32
_C = 4


def _matmul(x, w, bm):
    n, k = x.shape
    m = w.shape[1]

    def body(x_ref, w_ref, o_ref):
        o_ref[...] = jnp.dot(x_ref[...], w_ref[...],
                             preferred_element_type=jnp.float32)

    return pl.pallas_call(
        body,
        grid=(n // bm,),
        in_specs=[pl.BlockSpec((bm, k), lambda i: (i, 0)),
                  pl.BlockSpec((k, m), lambda i: (0, 0))],
        out_specs=pl.BlockSpec((bm, m), lambda i: (i, 0)),
        out_shape=jax.ShapeDtypeStruct((n, m), jnp.float32),
    )(x, w)


def _gat_sc_layer(NP, D, nheads, F, GW, apply_elu):
    npw = NP // _NW
    nchunks = npw // _C
    E = _C * D
    outw = nheads * F
    scol0 = nheads * F
    nacc = F // 16
    mesh = plsc.VectorSubcoreMesh(core_axis_name="c", subcore_axis_name="s")

    @functools.partial(
        pl.kernel,
        mesh=mesh,
        compiler_params=pltpu.CompilerParams(use_tc_tiling_on_sc=False,
                                             needs_layout_passes=False),
        out_type=jax.ShapeDtypeStruct((NP, outw), jnp.float32),
        scratch_types=[
            pltpu.VMEM((E,), jnp.int32),
            pltpu.VMEM((E,), jnp.int32),
            pltpu.VMEM((E, GW), jnp.float32),
            pltpu.VMEM((E, GW), jnp.float32),
            pltpu.VMEM((_C, outw), jnp.float32),
            pltpu.VMEM_SHARED((NP, GW), jnp.float32),
            pltpu.SemaphoreType.DMA,
            pltpu.SemaphoreType.DMA,
        ],
    )
    def k(adjf, g, out, idx0_v, idx1_v, rows0_v, rows1_v, o_v, gs, sem0, sem1):
        sid = lax.axis_index("s")
        wid = sid * 2 + lax.axis_index("c")
        base = wid * npw

        rpt = NP // 16
        pltpu.sync_copy(g.at[pl.ds(sid * rpt, rpt)], gs.at[pl.ds(sid * rpt, rpt)])
        plsc.subcore_barrier()

        def start_gather(c, idx_v, rows_v, sem):
            nb = base + c * _C
            pltpu.sync_copy(adjf.at[pl.ds(nb * D, E)], idx_v)
            pltpu.make_async_copy(gs.at[idx_v], rows_v, sem).start()

        def compute(c, rows_v):
            nb = base + c * _C

            def node_body(i, carry2):
                r0 = i * D
                stv = rows_v[r0, pl.ds(scol0, 16)]
                for h in range(nheads):
                    scol = scol0 + 2 * h
                    it = lax.iota(jnp.int32, 16)
                    cs = jnp.full((16,), scol, jnp.int32)
                    s0 = plsc.load_gather(rows_v, [r0 + it, cs])
                    s1 = plsc.load_gather(rows_v, [r0 + 16 + it, cs])
                    t = jnp.broadcast_to(stv[2 * h + 1], (16,))
                    e0 = s0 + t
                    e1 = s1 + t
                    e0 = jnp.maximum(e0, 0.2 * e0)
                    e1 = jnp.maximum(e1, 0.2 * e1)
                    m = jnp.maximum(jnp.max(e0), jnp.max(e1))
                    p0 = jnp.exp(e0 - m)
                    p1 = jnp.exp(e1 - m)
                    z = jnp.sum(p0) + jnp.sum(p1)
                    zrv = 1.0 / jnp.broadcast_to(z, (16,))
                    acc_a = [jnp.zeros((16,), jnp.float32) for _ in range(nacc)]
                    acc_b = [jnp.zeros((16,), jnp.float32) for _ in range(nacc)]
                    for d in range(0, D, 2):
                        pa = jnp.broadcast_to((p0 if d < 16 else p1)[d % 16], (16,))
                        pb = jnp.broadcast_to((p0 if d + 1 < 16 else p1)[(d + 1) % 16], (16,))
                        for fg in range(nacc):
                            col = h * F + fg * 16
                            acc_a[fg] = acc_a[fg] + pa * rows_v[r0 + d, pl.ds(col, 16)]
                            acc_b[fg] = acc_b[fg] + pb * rows_v[r0 + d + 1, pl.ds(col, 16)]
                    for fg in range(nacc):
                        acc = (acc_a[fg] + acc_b[fg]) * zrv
                        if apply_elu:
                            acc = jnp.where(acc > 0.0, acc, jnp.exp(acc) - 1.0)
                        o_v[i, pl.ds(h * F + fg * 16, 16)] = acc
                return carry2

            lax.fori_loop(0, _C, node_body, 0)
            pltpu.sync_copy(o_v, out.at[pl.ds(nb, _C)])

        npairs = nchunks // 2
        start_gather(0, idx0_v, rows0_v, sem0)

        def pair_body(gi, carry):
            a = 2 * gi
            start_gather(a + 1, idx1_v, rows1_v, sem1)
            pltpu.make_async_copy(gs.at[idx0_v], rows0_v, sem0).wait()
            compute(a, rows0_v)

            @pl.when(gi < npairs - 1)
            def _():
                start_gather(a + 2, idx0_v, rows0_v, sem0)

            pltpu.make_async_copy(gs.at[idx1_v], rows1_v, sem1).wait()
            compute(a + 1, rows1_v)
            return carry

        lax.fori_loop(0, npairs, pair_body, 0)

    return k


def _elu_logsoftmax(zin, n_valid):
    NPl, cls = zin.shape

    def body(z_ref, o_ref):
        zz = z_ref[...]
        x = jnp.where(zz > 0.0, zz, jnp.exp(zz) - 1.0)
        valid = lax.broadcasted_iota(jnp.int32, (NPl, cls), 0) < n_valid
        xm = jnp.where(valid, x, -jnp.inf)
        mx = jnp.max(xm, axis=0, keepdims=True)
        se = jnp.sum(jnp.exp(xm - mx), axis=0, keepdims=True)
        o_ref[...] = x - (mx + jnp.log(se))

    return pl.pallas_call(
        body,
        out_shape=jax.ShapeDtypeStruct((NPl, cls), jnp.float32),
    )(zin)


def kernel(embedding, adj, W_heads, a_heads, W_out, a_out):
    bs, N, nfeat = embedding.shape
    nheads, _, nhid = W_heads.shape
    D = adj.shape[2]
    nclass = W_out.shape[1]
    NP = -(-N // 1024) * 1024

    x = embedding.reshape(N, nfeat)
    xp = jnp.pad(x, ((0, NP - N), (0, 0)))
    adjf = jnp.pad(adj.reshape(N, D), ((0, NP - N), (0, 0))).reshape(NP * D)

    Wc = jnp.swapaxes(W_heads, 0, 1).reshape(nfeat, nheads * nhid)
    a1 = a_heads[:, :nhid, 0]
    a2 = a_heads[:, nhid:, 0]
    sW = jnp.einsum('hfk,hk->fh', W_heads, a1)
    tW = jnp.einsum('hfk,hk->fh', W_heads, a2)
    stW = jnp.stack([sW, tW], axis=2).reshape(nfeat, 2 * nheads)
    GW1 = 144
    M1 = jnp.concatenate(
        [Wc, stW,
         jnp.zeros((nfeat, GW1 - nheads * nhid - 2 * nheads), jnp.float32)],
        axis=1)
    G1 = _matmul(xp, M1, 512)

    l1 = _gat_sc_layer(NP, D, nheads, nhid, GW1, True)
    x1 = l1(adjf, G1)

    GW2 = 48
    M2 = jnp.concatenate(
        [W_out, W_out @ a_out[:nclass], W_out @ a_out[nclass:],
         jnp.zeros((nheads * nhid, GW2 - nclass - 2), jnp.float32)],
        axis=1)
    G2 = _matmul(x1, M2, 512)

    l2 = _gat_sc_layer(NP, D, 1, nclass, GW2, False)
    z = l2(adjf, G2)

    out = _elu_logsoftmax(z, N)
    return out[:N].reshape(bs, N, nclass)


# speedup vs baseline: 29.5665x; 1.2414x over previous
import functools

import jax
import jax.numpy as jnp
from jax import lax
from jax.experimental import pallas as pl
from jax.experimental.pallas import tpu as pltpu
from jax.experimental.pallas import tpu_sc as plsc

_NW = 32
_C = 4


def _matmul(x, w, bm):
    n, k = x.shape
    m = w.shape[1]

    def body(x_ref, w_ref, o_ref):
        o_ref[...] = jnp.dot(x_ref[...], w_ref[...],
                             preferred_element_type=jnp.float32)

    return pl.pallas_call(
        body,
        grid=(n // bm,),
        in_specs=[pl.BlockSpec((bm, k), lambda i: (i, 0)),
                  pl.BlockSpec((k, m), lambda i: (0, 0))],
        out_specs=pl.BlockSpec((bm, m), lambda i: (i, 0)),
        out_shape=jax.ShapeDtypeStruct((n, m), jnp.float32),
    )(x, w)


def _gat_sc_layer(NP, D, nheads, F, GW, apply_elu):
    npw = NP // _NW
    nchunks = npw // _C
    E = _C * D
    outw = nheads * F
    scol0 = nheads * F
    nacc = F // 16
    mesh = plsc.VectorSubcoreMesh(core_axis_name="c", subcore_axis_name="s")

    @functools.partial(
        pl.kernel,
        mesh=mesh,
        compiler_params=pltpu.CompilerParams(use_tc_tiling_on_sc=False,
                                             needs_layout_passes=False),
        out_type=jax.ShapeDtypeStruct((NP, outw), jnp.float32),
        scratch_types=[
            pltpu.VMEM((E,), jnp.int32),
            pltpu.VMEM((E,), jnp.int32),
            pltpu.VMEM((E, GW), jnp.float32),
            pltpu.VMEM((E, GW), jnp.float32),
            pltpu.VMEM((_C, outw), jnp.float32),
            pltpu.VMEM_SHARED((NP, GW), jnp.float32),
            pltpu.SemaphoreType.DMA,
            pltpu.SemaphoreType.DMA,
            pltpu.SemaphoreType.DMA,
            pltpu.SemaphoreType.DMA,
        ],
    )
    def k(adjf, g, out, idx0_v, idx1_v, rows0_v, rows1_v, o_v, gs,
          sem0, sem1, isem0, isem1):
        sid = lax.axis_index("s")
        wid = sid * 2 + lax.axis_index("c")
        base = wid * npw

        rpt = NP // 16
        pltpu.sync_copy(g.at[pl.ds(sid * rpt, rpt)], gs.at[pl.ds(sid * rpt, rpt)])
        plsc.subcore_barrier()

        def start_idx(c, idx_v, isem):
            nb = base + c * _C
            pltpu.make_async_copy(
                adjf.at[pl.ds(nb * D, E)], idx_v, isem).start()

        def wait_idx(idx_v, isem):
            pltpu.make_async_copy(adjf.at[pl.ds(0, E)], idx_v, isem).wait()

        def start_gather(idx_v, rows_v, sem):
            pltpu.make_async_copy(gs.at[idx_v], rows_v, sem).start()

        def wait_gather(idx_v, rows_v, sem):
            pltpu.make_async_copy(gs.at[idx_v], rows_v, sem).wait()

        def compute(c, rows_v):
            nb = base + c * _C

            def node_body(i, carry2):
                r0 = i * D
                stv = rows_v[r0, pl.ds(scol0, 16)]
                for h in range(nheads):
                    scol = scol0 + 2 * h
                    it = lax.iota(jnp.int32, 16)
                    cs = jnp.full((16,), scol, jnp.int32)
                    s0 = plsc.load_gather(rows_v, [r0 + it, cs])
                    s1 = plsc.load_gather(rows_v, [r0 + 16 + it, cs])
                    t = jnp.broadcast_to(stv[2 * h + 1], (16,))
                    e0 = s0 + t
                    e1 = s1 + t
                    e0 = jnp.maximum(e0, 0.2 * e0)
                    e1 = jnp.maximum(e1, 0.2 * e1)
                    m = jnp.maximum(jnp.max(e0), jnp.max(e1))
                    p0 = jnp.exp(e0 - m)
                    p1 = jnp.exp(e1 - m)
                    z = jnp.sum(p0) + jnp.sum(p1)
                    zrv = 1.0 / jnp.broadcast_to(z, (16,))
                    acc_a = [jnp.zeros((16,), jnp.float32) for _ in range(nacc)]
                    acc_b = [jnp.zeros((16,), jnp.float32) for _ in range(nacc)]
                    for d in range(0, D, 2):
                        pa = jnp.broadcast_to((p0 if d < 16 else p1)[d % 16], (16,))
                        pb = jnp.broadcast_to((p0 if d + 1 < 16 else p1)[(d + 1) % 16], (16,))
                        for fg in range(nacc):
                            col = h * F + fg * 16
                            acc_a[fg] = acc_a[fg] + pa * rows_v[r0 + d, pl.ds(col, 16)]
                            acc_b[fg] = acc_b[fg] + pb * rows_v[r0 + d + 1, pl.ds(col, 16)]
                    for fg in range(nacc):
                        acc = (acc_a[fg] + acc_b[fg]) * zrv
                        if apply_elu:
                            acc = jnp.where(acc > 0.0, acc, jnp.exp(acc) - 1.0)
                        o_v[i, pl.ds(h * F + fg * 16, 16)] = acc
                return carry2

            lax.fori_loop(0, _C, node_body, 0)
            pltpu.sync_copy(o_v, out.at[pl.ds(nb, _C)])

        npairs = nchunks // 2
        start_idx(0, idx0_v, isem0)
        start_idx(1, idx1_v, isem1)
        wait_idx(idx0_v, isem0)
        start_gather(idx0_v, rows0_v, sem0)

        def pair_body(gi, carry):
            a = 2 * gi
            wait_idx(idx1_v, isem1)
            start_gather(idx1_v, rows1_v, sem1)
            wait_gather(idx0_v, rows0_v, sem0)

            @pl.when(gi < npairs - 1)
            def _():
                start_idx(a + 2, idx0_v, isem0)

            compute(a, rows0_v)

            @pl.when(gi < npairs - 1)
            def _():
                wait_idx(idx0_v, isem0)
                start_gather(idx0_v, rows0_v, sem0)

            wait_gather(idx1_v, rows1_v, sem1)

            @pl.when(gi < npairs - 1)
            def _():
                start_idx(a + 3, idx1_v, isem1)

            compute(a + 1, rows1_v)
            return carry

        lax.fori_loop(0, npairs, pair_body, 0)

    return k


def _elu_logsoftmax(zin, n_valid):
    NPl, cls = zin.shape

    def body(z_ref, o_ref):
        zz = z_ref[...]
        x = jnp.where(zz > 0.0, zz, jnp.exp(zz) - 1.0)
        valid = lax.broadcasted_iota(jnp.int32, (NPl, cls), 0) < n_valid
        xm = jnp.where(valid, x, -jnp.inf)
        mx = jnp.max(xm, axis=0, keepdims=True)
        se = jnp.sum(jnp.exp(xm - mx), axis=0, keepdims=True)
        o_ref[...] = x - (mx + jnp.log(se))

    return pl.pallas_call(
        body,
        out_shape=jax.ShapeDtypeStruct((NPl, cls), jnp.float32),
    )(zin)


def kernel(embedding, adj, W_heads, a_heads, W_out, a_out):
    bs, N, nfeat = embedding.shape
    nheads, _, nhid = W_heads.shape
    D = adj.shape[2]
    nclass = W_out.shape[1]
    NP = -(-N // 1024) * 1024

    x = embedding.reshape(N, nfeat)
    xp = jnp.pad(x, ((0, NP - N), (0, 0)))
    adjf = jnp.pad(adj.reshape(N, D), ((0, NP - N), (0, 0))).reshape(NP * D)

    Wc = jnp.swapaxes(W_heads, 0, 1).reshape(nfeat, nheads * nhid)
    a1 = a_heads[:, :nhid, 0]
    a2 = a_heads[:, nhid:, 0]
    sW = jnp.einsum('hfk,hk->fh', W_heads, a1)
    tW = jnp.einsum('hfk,hk->fh', W_heads, a2)
    stW = jnp.stack([sW, tW], axis=2).reshape(nfeat, 2 * nheads)
    GW1 = 144
    M1 = jnp.concatenate(
        [Wc, stW,
         jnp.zeros((nfeat, GW1 - nheads * nhid - 2 * nheads), jnp.float32)],
        axis=1)
    G1 = _matmul(xp, M1, 512)

    l1 = _gat_sc_layer(NP, D, nheads, nhid, GW1, True)
    x1 = l1(adjf, G1)

    GW2 = 48
    M2 = jnp.concatenate(
        [W_out, W_out @ a_out[:nclass], W_out @ a_out[nclass:],
         jnp.zeros((nheads * nhid, GW2 - nclass - 2), jnp.float32)],
        axis=1)
    G2 = _matmul(x1, M2, 512)

    l2 = _gat_sc_layer(NP, D, 1, nclass, GW2, False)
    z = l2(adjf, G2)

    out = _elu_logsoftmax(z, N)
    return out[:N].reshape(bs, N, nclass)
